# baseline (speedup 1.0000x reference)
import jax
import jax.numpy as jnp
import numpy as np
from jax.experimental import pallas as pl

B = 64
DEPTH = 10
M = 2 ** DEPTH - 1
D_WORD = 128
H = 128
T = 8


def _bitrev(nbits):
    k = np.arange(2 ** nbits)
    r = np.zeros_like(k)
    for b in range(nbits):
        r |= ((k >> b) & 1) << (nbits - 1 - b)
    return r


_PERM = np.concatenate(
    [(2 ** d - 1) + _bitrev(d) for d in range(DEPTH - 1, -1, -1)])
_OFF = {d: sum(2 ** dd for dd in range(DEPTH - 1, d, -1))
        for d in range(DEPTH)}


def _sig(x):
    return 0.5 * jnp.tanh(0.5 * x) + 0.5


def _lstm_block(nf_ref, w_iou, b_iou, u_iou, u_f, b_f):
    acc = jnp.zeros((T, H), jnp.float32)
    h_prev = None
    c_prev = None
    for d in range(DEPTH - 1, -1, -1):
        n = 2 ** d
        off = _OFF[d]
        feat = nf_ref[:, off:off + n, :].reshape(T * n, D_WORD)
        iou = jnp.dot(feat, w_iou, preferred_element_type=jnp.float32) + b_iou
        if h_prev is None:
            c = _sig(iou[:, :H]) * jnp.tanh(iou[:, 2 * H:])
        else:
            f = _sig(jnp.dot(h_prev.astype(jnp.bfloat16), u_f,
                             preferred_element_type=jnp.float32) + b_f)
            fc = f * c_prev
            hp = h_prev.reshape(T, 2, n, H)
            fcp = fc.reshape(T, 2, n, H)
            h_sum = (hp[:, 0] + hp[:, 1]).reshape(T * n, H)
            c_sum = (fcp[:, 0] + fcp[:, 1]).reshape(T * n, H)
            iou = iou + jnp.dot(h_sum.astype(jnp.bfloat16), u_iou,
                                preferred_element_type=jnp.float32)
            c = _sig(iou[:, :H]) * jnp.tanh(iou[:, 2 * H:]) + c_sum
        h = _sig(iou[:, H:2 * H]) * jnp.tanh(c)
        acc = acc + h.reshape(T, n, H).sum(axis=1)
        h_prev, c_prev = h, c
    return acc


def _both_body(nf1_ref, nf2_ref,
               w1_ref, bi1_ref, ui1_ref, uf1_ref, bf1_ref,
               w2_ref, bi2_ref, ui2_ref, uf2_ref, bf2_ref,
               out1_ref, out2_ref):
    out1_ref[...] = _lstm_block(nf1_ref, w1_ref[...], bi1_ref[...],
                                ui1_ref[...], uf1_ref[...], bf1_ref[...])
    out2_ref[...] = _lstm_block(nf2_ref, w2_ref[...], bi2_ref[...],
                                ui2_ref[...], uf2_ref[...], bf2_ref[...])


def _both_lstms(nf1p, nf2p, W1, bi1, Ui1, Uf1, bf1, W2, bi2, Ui2, Uf2, bf2):
    feat_spec = pl.BlockSpec((T, M, D_WORD), lambda i: (i, 0, 0))
    w_specs = [
        pl.BlockSpec((D_WORD, 3 * H), lambda i: (0, 0)),
        pl.BlockSpec((1, 3 * H), lambda i: (0, 0)),
        pl.BlockSpec((H, 3 * H), lambda i: (0, 0)),
        pl.BlockSpec((H, H), lambda i: (0, 0)),
        pl.BlockSpec((1, H), lambda i: (0, 0)),
    ]
    bf = jnp.bfloat16
    return pl.pallas_call(
        _both_body,
        grid=(B // T,),
        in_specs=[feat_spec, feat_spec] + w_specs + w_specs,
        out_specs=[pl.BlockSpec((T, H), lambda i: (i, 0))] * 2,
        out_shape=[jax.ShapeDtypeStruct((B, H), jnp.float32)] * 2,
    )(nf1p, nf2p,
      W1.astype(bf), bi1.reshape(1, 3 * H), Ui1.astype(bf), Uf1.astype(bf),
      bf1.reshape(1, H),
      W2.astype(bf), bi2.reshape(1, 3 * H), Ui2.astype(bf), Uf2.astype(bf),
      bf2.reshape(1, H))


def _head_body(h1_ref, h2_ref, wff_ref, bff_ref, out_ref):
    inv_m = 1.0 / M
    mf1 = jnp.maximum(h1_ref[...] * inv_m, 0.0)
    mf2 = jnp.maximum(h2_ref[...] * inv_m, 0.0)
    w = wff_ref[...]
    dense = (jnp.dot(mf1, w[:H], preferred_element_type=jnp.float32)
             + jnp.dot(mf2, w[H:], preferred_element_type=jnp.float32)
             + bff_ref[...])
    act = jnp.where(dense >= 0, dense, 0.01 * dense)
    col = jax.lax.broadcasted_iota(jnp.int32, act.shape, 1)
    act = jnp.where(col < 2, act, -jnp.inf)
    m = jnp.max(act, axis=1, keepdims=True)
    e = jnp.exp(act - m)
    out_ref[...] = e / jnp.sum(e, axis=1, keepdims=True)


def _head(hsum1, hsum2, W_ff, b_ff):
    W_pad = jnp.zeros((2 * H, 128), jnp.float32).at[:, :2].set(W_ff)
    b_pad = jnp.zeros((1, 128), jnp.float32).at[:, :2].set(b_ff)
    out = pl.pallas_call(
        _head_body,
        out_shape=jax.ShapeDtypeStruct((B, 128), jnp.float32),
    )(hsum1, hsum2, W_pad, b_pad)
    return out[:, :2]


def kernel(node_feat1, node_feat2, mask1, mask2,
           W_iou1, b_iou1, U_iou1, U_f1, b_f1,
           W_iou2, b_iou2, U_iou2, U_f2, b_f2,
           W_ff, b_ff, parent, level, graph_id):
    nf1p = node_feat1.astype(jnp.bfloat16).reshape(B, M, D_WORD)[:, _PERM, :]
    nf2p = node_feat2.astype(jnp.bfloat16).reshape(B, M, D_WORD)[:, _PERM, :]
    hsum1, hsum2 = _both_lstms(nf1p, nf2p,
                               W_iou1, b_iou1, U_iou1, U_f1, b_f1,
                               W_iou2, b_iou2, U_iou2, U_f2, b_f2)
    return _head(hsum1, hsum2, W_ff, b_ff)


# speedup vs baseline: 1.0191x; 1.0191x over previous
import jax
import jax.numpy as jnp
import numpy as np
from jax.experimental import pallas as pl
from jax.experimental.pallas import tpu as pltpu

B = 64
DEPTH = 10
M = 2 ** DEPTH - 1
D_WORD = 128
H = 128
T = 8


def _bitrev(nbits):
    k = np.arange(2 ** nbits)
    r = np.zeros_like(k)
    for b in range(nbits):
        r |= ((k >> b) & 1) << (nbits - 1 - b)
    return r


_PERM = np.concatenate(
    [(2 ** d - 1) + _bitrev(d) for d in range(DEPTH - 1, -1, -1)])
_OFF = {d: sum(2 ** dd for dd in range(DEPTH - 1, d, -1))
        for d in range(DEPTH)}


def _sig(x):
    return 0.5 * jnp.tanh(0.5 * x) + 0.5


def _lstm_block(nf_ref, w_iou, b_iou, u_iou, u_f, b_f):
    acc = jnp.zeros((T, H), jnp.float32)
    h_prev = None
    c_prev = None
    for d in range(DEPTH - 1, -1, -1):
        n = 2 ** d
        off = _OFF[d]
        feat = nf_ref[:, off:off + n, :].reshape(T * n, D_WORD)
        iou = jnp.dot(feat, w_iou, preferred_element_type=jnp.float32) + b_iou
        if h_prev is None:
            c = _sig(iou[:, :H]) * jnp.tanh(iou[:, 2 * H:])
        else:
            f = _sig(jnp.dot(h_prev.astype(jnp.bfloat16), u_f,
                             preferred_element_type=jnp.float32) + b_f)
            fc = f * c_prev
            hp = h_prev.reshape(T, 2, n, H)
            fcp = fc.reshape(T, 2, n, H)
            h_sum = (hp[:, 0] + hp[:, 1]).reshape(T * n, H)
            c_sum = (fcp[:, 0] + fcp[:, 1]).reshape(T * n, H)
            iou = iou + jnp.dot(h_sum.astype(jnp.bfloat16), u_iou,
                                preferred_element_type=jnp.float32)
            c = _sig(iou[:, :H]) * jnp.tanh(iou[:, 2 * H:]) + c_sum
        h = _sig(iou[:, H:2 * H]) * jnp.tanh(c)
        acc = acc + h.reshape(T, n, H).sum(axis=1)
        h_prev, c_prev = h, c
    return acc


def _lstm1_body(nf_ref, w_iou_ref, b_iou_ref, u_iou_ref, u_f_ref, b_f_ref,
                out_ref):
    out_ref[...] = _lstm_block(nf_ref, w_iou_ref[...], b_iou_ref[...],
                               u_iou_ref[...], u_f_ref[...], b_f_ref[...])


_W_SPECS = None


def _tree_lstm(nf_perm, W_iou, b_iou, U_iou, U_f, b_f):
    in_specs = [
        pl.BlockSpec((T, M, D_WORD), lambda i: (i, 0, 0)),
        pl.BlockSpec((D_WORD, 3 * H), lambda i: (0, 0)),
        pl.BlockSpec((1, 3 * H), lambda i: (0, 0)),
        pl.BlockSpec((H, 3 * H), lambda i: (0, 0)),
        pl.BlockSpec((H, H), lambda i: (0, 0)),
        pl.BlockSpec((1, H), lambda i: (0, 0)),
    ]
    return pl.pallas_call(
        _lstm1_body,
        grid=(B // T,),
        in_specs=in_specs,
        out_specs=pl.BlockSpec((T, H), lambda i: (i, 0)),
        out_shape=jax.ShapeDtypeStruct((B, H), jnp.float32),
    )(nf_perm, W_iou.astype(jnp.bfloat16), b_iou.reshape(1, 3 * H),
      U_iou.astype(jnp.bfloat16), U_f.astype(jnp.bfloat16),
      b_f.reshape(1, H))


def _lstm2_head_body(nf_ref, w_iou_ref, b_iou_ref, u_iou_ref, u_f_ref,
                     b_f_ref, h1_ref, wff_ref, bff_ref, out_ref, acc_ref):
    i = pl.program_id(0)
    acc = _lstm_block(nf_ref, w_iou_ref[...], b_iou_ref[...],
                      u_iou_ref[...], u_f_ref[...], b_f_ref[...])
    acc_ref[pl.ds(i * T, T), :] = acc

    @pl.when(i == (B // T) - 1)
    def _():
        inv_m = 1.0 / M
        mf1 = jnp.maximum(h1_ref[...] * inv_m, 0.0)
        mf2 = jnp.maximum(acc_ref[...] * inv_m, 0.0)
        w = wff_ref[...]
        dense = (jnp.dot(mf1, w[:H], preferred_element_type=jnp.float32)
                 + jnp.dot(mf2, w[H:], preferred_element_type=jnp.float32)
                 + bff_ref[...])
        act = jnp.where(dense >= 0, dense, 0.01 * dense)
        col = jax.lax.broadcasted_iota(jnp.int32, act.shape, 1)
        act = jnp.where(col < 2, act, -jnp.inf)
        mx = jnp.max(act, axis=1, keepdims=True)
        e = jnp.exp(act - mx)
        out_ref[...] = e / jnp.sum(e, axis=1, keepdims=True)


def _lstm2_with_head(nf_perm, W_iou, b_iou, U_iou, U_f, b_f,
                     hsum1, W_ff, b_ff):
    W_pad = jnp.zeros((2 * H, 128), jnp.float32).at[:, :2].set(W_ff)
    b_pad = jnp.zeros((1, 128), jnp.float32).at[:, :2].set(b_ff)
    in_specs = [
        pl.BlockSpec((T, M, D_WORD), lambda i: (i, 0, 0)),
        pl.BlockSpec((D_WORD, 3 * H), lambda i: (0, 0)),
        pl.BlockSpec((1, 3 * H), lambda i: (0, 0)),
        pl.BlockSpec((H, 3 * H), lambda i: (0, 0)),
        pl.BlockSpec((H, H), lambda i: (0, 0)),
        pl.BlockSpec((1, H), lambda i: (0, 0)),
        pl.BlockSpec((B, H), lambda i: (0, 0)),
        pl.BlockSpec((2 * H, 128), lambda i: (0, 0)),
        pl.BlockSpec((1, 128), lambda i: (0, 0)),
    ]
    out = pl.pallas_call(
        _lstm2_head_body,
        grid=(B // T,),
        in_specs=in_specs,
        out_specs=pl.BlockSpec((B, 128), lambda i: (0, 0)),
        out_shape=jax.ShapeDtypeStruct((B, 128), jnp.float32),
        scratch_shapes=[pltpu.VMEM((B, H), jnp.float32)],
    )(nf_perm, W_iou.astype(jnp.bfloat16), b_iou.reshape(1, 3 * H),
      U_iou.astype(jnp.bfloat16), U_f.astype(jnp.bfloat16),
      b_f.reshape(1, H), hsum1, W_pad, b_pad)
    return out[:, :2]


def kernel(node_feat1, node_feat2, mask1, mask2,
           W_iou1, b_iou1, U_iou1, U_f1, b_f1,
           W_iou2, b_iou2, U_iou2, U_f2, b_f2,
           W_ff, b_ff, parent, level, graph_id):
    nf1p = node_feat1.astype(jnp.bfloat16).reshape(B, M, D_WORD)[:, _PERM, :]
    nf2p = node_feat2.astype(jnp.bfloat16).reshape(B, M, D_WORD)[:, _PERM, :]
    hsum1 = _tree_lstm(nf1p, W_iou1, b_iou1, U_iou1, U_f1, b_f1)
    return _lstm2_with_head(nf2p, W_iou2, b_iou2, U_iou2, U_f2, b_f2,
                            hsum1, W_ff, b_ff)
